# baseline (speedup 1.0000x reference)
import functools

import jax
import jax.numpy as jnp
from jax import lax
from jax.experimental import pallas as pl
from jax.experimental.pallas import tpu as pltpu
from jax.experimental.pallas import tpu_sc as plsc

_LANES = 16
_CHUNK_ROWS = 16
_SC_BATCHES = 1


def _tc_body(x_ref, pos_ref, out_ref):
    out_ref[...] = x_ref[...] + pos_ref[...][None]


def _tc_add(x, pos_table):
    B, S, D = x.shape
    return pl.pallas_call(
        _tc_body,
        grid=(1, B),
        in_specs=[
            pl.BlockSpec((1, S, D), lambda i, b: (b, i, 0)),
            pl.BlockSpec((S, D), lambda i, b: (i, 0)),
        ],
        out_specs=pl.BlockSpec((1, S, D), lambda i, b: (b, i, 0)),
        out_shape=jax.ShapeDtypeStruct((B, S, D), x.dtype),
    )(x, pos_table)


def _sc_add(x, pos_table):
    B, S, D = x.shape
    info = plsc.get_sparse_core_info()
    NC, NS = info.num_cores, info.num_subcores
    NW = NC * NS
    SPW = S // NW
    CH = _CHUNK_ROWS
    cps = SPW // CH
    N = B * cps
    VECS = D // _LANES

    mesh = plsc.VectorSubcoreMesh(core_axis_name="c", subcore_axis_name="s")

    @functools.partial(
        pl.kernel,
        mesh=mesh,
        out_type=jax.ShapeDtypeStruct((B, S, D), jnp.float32),
        scratch_types=[
            pltpu.VMEM((SPW, D), jnp.float32),
            pltpu.VMEM((CH, D), jnp.float32),
            pltpu.VMEM((CH, D), jnp.float32),
            pltpu.SemaphoreType.DMA,
            pltpu.SemaphoreType.DMA,
            pltpu.SemaphoreType.DMA,
            pltpu.SemaphoreType.DMA,
        ],
    )
    def k(x_hbm, pos_hbm, out_hbm, pos_v, xbuf0, xbuf1, si0, si1, so0, so1):
        wid = lax.axis_index("s") * NC + lax.axis_index("c")
        s_base = wid * SPW
        pltpu.sync_copy(pos_hbm.at[pl.ds(s_base, SPW)], pos_v)

        bufs = (xbuf0, xbuf1)
        sin = (si0, si1)
        sout = (so0, so1)

        def src(ci):
            return x_hbm.at[ci // cps, pl.ds(s_base + (ci % cps) * CH, CH)]

        def dst(ci):
            return out_hbm.at[ci // cps, pl.ds(s_base + (ci % cps) * CH, CH)]

        def compute(buf, ci):
            prow0 = (ci % cps) * CH

            @plsc.parallel_loop(0, CH)
            def _(r):
                prow = prow0 + r
                for c in range(VECS):
                    pv = pos_v[prow, pl.ds(c * _LANES, _LANES)]
                    plsc.addupdate(buf.at[r, pl.ds(c * _LANES, _LANES)], pv)

        pltpu.async_copy(src(0), bufs[0], sin[0])

        def pair_body(g, carry):
            for b in range(2):
                ci = 2 * g + b

                @pl.when(ci >= 1)
                def _():
                    pltpu.make_async_copy(bufs[1 - b], dst(ci), sout[1 - b]).wait()

                @pl.when(ci + 1 < N)
                def _():
                    pltpu.async_copy(src(ci + 1), bufs[1 - b], sin[1 - b])

                pltpu.make_async_copy(src(ci), bufs[b], sin[b]).wait()
                compute(bufs[b], ci)
                pltpu.async_copy(bufs[b], dst(ci), sout[b])
            return carry

        lax.fori_loop(0, N // 2, pair_body, 0)
        pltpu.make_async_copy(bufs[1], dst(N - 1), sout[1]).wait()

    return k(x, pos_table)


def kernel(x, pos_table):
    B, S, D = x.shape
    bt = B - _SC_BATCHES
    out_tc = _tc_add(x[:bt], pos_table)
    out_sc = _sc_add(x[bt:], pos_table)
    return jnp.concatenate([out_tc, out_sc], axis=0)


# speedup vs baseline: 3.7531x; 3.7531x over previous
import jax
import jax.numpy as jnp
from jax.experimental import pallas as pl

_BLOCK_S = 2048


def _body(x_ref, pos_ref, out_ref):
    out_ref[...] = x_ref[...] + pos_ref[...][None]


def kernel(x, pos_table):
    B, S, D = x.shape
    grid = (S // _BLOCK_S, B)
    return pl.pallas_call(
        _body,
        grid=grid,
        in_specs=[
            pl.BlockSpec((1, _BLOCK_S, D), lambda i, b: (b, i, 0)),
            pl.BlockSpec((_BLOCK_S, D), lambda i, b: (i, 0)),
        ],
        out_specs=pl.BlockSpec((1, _BLOCK_S, D), lambda i, b: (b, i, 0)),
        out_shape=jax.ShapeDtypeStruct((B, S, D), x.dtype),
    )(x, pos_table)
